# baseline (speedup 1.0000x reference)
import functools

import jax
import jax.numpy as jnp
from jax import lax
from jax.experimental import pallas as pl
from jax.experimental.pallas import tpu as pltpu
from jax.experimental.pallas import tpu_sc as plsc

N = 10000
E = 160000
VIS = 512
H = 512
BN = 1000
GRID = N // BN

NC = 2
NS = 16
NW = NC * NS
TASK = 40
NTASK = N // TASK
GB = 96
EPAD = 256
RPPAD = 64


def _f32(x):
    return x.astype(jnp.float32)


def _enc_body(feat, goal, info, w1a, w1b, w1c, b1, w2, b2, w3, b3, out):
    x = jnp.dot(feat[...], w1a[...], preferred_element_type=jnp.float32)
    x += jnp.dot(goal[...], w1b[...], preferred_element_type=jnp.float32)
    x += jnp.dot(info[...], w1c[...], preferred_element_type=jnp.float32)
    x = jnp.maximum(x + b1[...], 0.0)
    x = jnp.maximum(jnp.dot(x, w2[...], preferred_element_type=jnp.float32) + b2[...], 0.0)
    out[...] = jnp.dot(x, w3[...], preferred_element_type=jnp.float32) + b3[...]


def _gat_pre_body(x, goal, info, wx, wg, wi, a2p, h_out, av_out):
    h = jnp.dot(x[...], wx[...], preferred_element_type=jnp.float32)
    h += jnp.dot(goal[...], wg[...], preferred_element_type=jnp.float32)
    h += jnp.dot(info[...], wi[...], preferred_element_type=jnp.float32)
    h_out[...] = h
    av_out[...] = jnp.dot(h, a2p[...], preferred_element_type=jnp.float32)


def _dec_body(x, goal, info, w1a, w1b, w1c, b1, w2, b2, w3p, b3p, out):
    y = jnp.dot(x[...], w1a[...], preferred_element_type=jnp.float32)
    y += jnp.dot(goal[...], w1b[...], preferred_element_type=jnp.float32)
    y += jnp.dot(info[...], w1c[...], preferred_element_type=jnp.float32)
    y = jnp.maximum(y + b1[...], 0.0)
    y = jnp.maximum(jnp.dot(y, w2[...], preferred_element_type=jnp.float32) + b2[...], 0.0)
    y = jnp.dot(y, w3p[...], preferred_element_type=jnp.float32) + b3p[...]
    out[...] = jax.nn.sigmoid(y)


def _row_spec(d):
    return pl.BlockSpec((BN, d), lambda i: (i, 0))


def _full_spec(shape):
    return pl.BlockSpec(shape, lambda i: tuple(0 for _ in shape))


def _enc_call(feat, goal, info, w1a, w1b, w1c, b1, w2, b2, w3, b3):
    return pl.pallas_call(
        _enc_body,
        grid=(GRID,),
        in_specs=[
            _row_spec(VIS), _row_spec(VIS), _row_spec(4),
            _full_spec((VIS, H)), _full_spec((VIS, H)), _full_spec((4, H)),
            _full_spec((1, H)),
            _full_spec((H, H)), _full_spec((1, H)),
            _full_spec((H, H)), _full_spec((1, H)),
        ],
        out_specs=_row_spec(H),
        out_shape=jax.ShapeDtypeStruct((N, H), jnp.float32),
    )(feat, goal, info, w1a, w1b, w1c, b1, w2, b2, w3, b3)


def _gat_pre_call(x, goal, info, wx, wg, wi, a2p):
    return pl.pallas_call(
        _gat_pre_body,
        grid=(GRID,),
        in_specs=[
            _row_spec(H), _row_spec(VIS), _row_spec(4),
            _full_spec((H, H)), _full_spec((VIS, H)), _full_spec((4, H)),
            _full_spec((H, 128)),
        ],
        out_specs=[_row_spec(H), _row_spec(128)],
        out_shape=[
            jax.ShapeDtypeStruct((N, H), jnp.float32),
            jax.ShapeDtypeStruct((N, 128), jnp.float32),
        ],
    )(x, goal, info, wx, wg, wi, a2p)


def _dec_call(x, goal, info, w1a, w1b, w1c, b1, w2, b2, w3p, b3p):
    return pl.pallas_call(
        _dec_body,
        grid=(GRID,),
        in_specs=[
            _row_spec(H), _row_spec(VIS), _row_spec(4),
            _full_spec((H, H)), _full_spec((VIS, H)), _full_spec((4, H)),
            _full_spec((1, H)),
            _full_spec((H, H)), _full_spec((1, H)),
            _full_spec((H, 128)), _full_spec((1, 128)),
        ],
        out_specs=_row_spec(128),
        out_shape=jax.ShapeDtypeStruct((N, 128), jnp.float32),
    )(x, goal, info, w1a, w1b, w1c, b1, w2, b2, w3p, b3p)


NACC = H // 16


def _sc_gat_body(h_hbm, asrc_hbm, adst_hbm, srcs_hbm, dsts_hbm, rp_hbm,
                 x_hbm,
                 rpv, evbuf, hp2, sbuf3, dbuf3, semI3,
                 cbuf0, asg0, adg0, rows0, semG0,
                 cbuf1, asg1, adg1, rows1, semG1):
    wid = lax.axis_index("s") * NC + lax.axis_index("c")
    slots = ((cbuf0, asg0, adg0, rows0, semG0),
             (cbuf1, asg1, adg1, rows1, semG1))

    zero16 = jnp.zeros((16,), jnp.float32)
    iota = lax.iota(jnp.int32, 16)
    lane0 = iota == 0
    zsel = jnp.zeros((16,), jnp.int32)

    def do_task(t):
        r0 = t * TASK

        def flush(cur, rs, acc):
            @pl.when(jnp.logical_and(cur >= r0, cur < r0 + TASK))
            def _():
                inv = 1.0 / jnp.full((16,), rs + 1e-16, jnp.float32)
                loc = cur - r0
                for c in range(NACC):
                    val = jnp.maximum(acc[c] * inv, 0.0)
                    hp2[pl.ds(loc * H + c * 16, 16)] = val

        def edge_step(rw, k, sk, ek, state):
            cur, rs, acc = state
            is_new = sk != cur

            @pl.when(is_new)
            def _():
                flush(cur, rs, acc)

            keepv = jnp.full((16,), jnp.where(is_new, 0.0, 1.0), jnp.float32)
            ekv = jnp.full((16,), ek, jnp.float32)
            nacc = tuple(acc[c] * keepv + ekv * rw[k, pl.ds(c * 16, 16)]
                         for c in range(NACC))
            nrs = jnp.where(is_new, ek, rs + ek)
            return sk, nrs, nacc

        pltpu.sync_copy(rp_hbm.at[pl.ds(r0, RPPAD)], rpv)

        def zrow(z, _):
            for c in range(NACC):
                hp2[pl.ds(z * H + c * 16, 16)] = zero16
            return 0
        lax.fori_loop(0, TASK, zrow, 0)

        p0 = rpv[pl.ds(0, 16)][0]
        p1 = rpv[pl.ds(TASK - 8, 16)][8]
        b0 = (p0 // 16) * 16
        nb = (p1 - b0 + GB - 1) // GB
        nb2 = ((nb + 1) // 2) * 2

        def issue_idx(g):
            islot = lax.rem(g, 3)
            @pl.when(g < nb2)
            def _():
                pos = b0 + g * GB
                pltpu.async_copy(srcs_hbm.at[pl.ds(pos, GB)],
                                 sbuf3.at[islot], semI3.at[islot])
                pltpu.async_copy(dsts_hbm.at[pl.ds(pos, GB)],
                                 dbuf3.at[islot], semI3.at[islot])

        def start_gather(g, s):
            cb, ag, dg, rw, sG = slots[s]
            islot = lax.rem(g, 3)
            @pl.when(g < nb2)
            def _():
                pltpu.make_async_copy(srcs_hbm.at[pl.ds(0, GB)],
                                      sbuf3.at[islot], semI3.at[islot]).wait()
                pltpu.make_async_copy(dsts_hbm.at[pl.ds(0, GB)],
                                      dbuf3.at[islot], semI3.at[islot]).wait()
                for j in range(GB // 16):
                    cb[pl.ds(j * 16, 16)] = jnp.minimum(
                        sbuf3[islot, pl.ds(j * 16, 16)], N - 1)
                pltpu.async_copy(h_hbm.at[dbuf3.at[islot]], rw, sG)
                pltpu.async_copy(asrc_hbm.at[cb], ag, sG)
                pltpu.async_copy(adst_hbm.at[dbuf3.at[islot]], dg, sG)

        def process(g, s, state):
            cb, ag, dg, rw, sG = slots[s]
            islot = lax.rem(g, 3)
            pltpu.make_async_copy(h_hbm.at[dbuf3.at[islot]], rw, sG).wait()
            pltpu.make_async_copy(asrc_hbm.at[cb], ag, sG).wait()
            pltpu.make_async_copy(adst_hbm.at[dbuf3.at[islot]], dg, sG).wait()
            for j in range(GB // 16):
                az = ag[pl.ds(j * 16, 16)] + dg[pl.ds(j * 16, 16)]
                evbuf[pl.ds(j * 16, 16)] = jnp.exp(
                    jnp.minimum(-az, -0.2 * az))

            def sub(j, st):
                sv16 = sbuf3[islot, pl.ds(j * 16, 16)]
                ev16 = evbuf[pl.ds(j * 16, 16)]
                for k16 in range(16):
                    st = edge_step(rw, j * 16 + k16, sv16[k16], ev16[k16], st)
                return st
            return lax.fori_loop(0, GB // 16, sub, state)

        issue_idx(0)
        issue_idx(1)
        issue_idx(2)
        start_gather(0, 0)

        def pair(m, state):
            g = m * 2
            start_gather(g + 1, 1)
            state = process(g, 0, state)
            issue_idx(g + 3)
            start_gather(g + 2, 0)
            state = process(g + 1, 1, state)
            issue_idx(g + 4)
            return state

        acc0 = tuple(zero16 for _ in range(NACC))
        state = lax.fori_loop(0, nb2 // 2, pair,
                              (jnp.int32(-1), jnp.float32(0.0), acc0))
        flush(state[0], state[1], state[2])

        pltpu.sync_copy(hp2, x_hbm.at[pl.ds(r0 * H, TASK * H)])

    nt_w = jnp.where(wid < (NTASK % NW), NTASK // NW + 1, NTASK // NW)

    def task_loop(i, _):
        do_task(wid + i * NW)
        return 0
    lax.fori_loop(0, nt_w, task_loop, 0)


def _sc_gat_call(h, asrc, adst, srcs_pad, dsts_pad, rp_pad):
    mesh = plsc.VectorSubcoreMesh(core_axis_name="c", subcore_axis_name="s")
    kfn = functools.partial(
        pl.kernel,
        out_type=jax.ShapeDtypeStruct((N * H,), jnp.float32),
        mesh=mesh,
        compiler_params=pltpu.CompilerParams(needs_layout_passes=False),
        scratch_types=[
            pltpu.VMEM((RPPAD,), jnp.int32),
            pltpu.VMEM((GB,), jnp.float32),
            pltpu.VMEM((TASK * H,), jnp.float32),
            pltpu.VMEM((3, GB), jnp.int32),
            pltpu.VMEM((3, GB), jnp.int32),
            pltpu.SemaphoreType.DMA((3,)),
        ] + 2 * [
            pltpu.VMEM((GB,), jnp.int32),
            pltpu.VMEM((GB,), jnp.float32),
            pltpu.VMEM((GB,), jnp.float32),
            pltpu.VMEM((GB, H), jnp.float32),
            pltpu.SemaphoreType.DMA,
        ],
    )(_sc_gat_body)
    return kfn(h, asrc, adst, srcs_pad, dsts_pad, rp_pad).reshape(N, H)


def kernel(feat, goal_feat, info_feat, adj,
           enc_W1, enc_b1, enc_W2, enc_b2, enc_W3, enc_b3,
           val_W1, val_b1, val_W2, val_b2, val_W3, val_b3,
           gat1_W, gat1_a, gat2_W, gat2_a, gat3_W, gat3_a,
           gat4_W, gat4_a, gat5_W, gat5_a):
    feat = _f32(feat); goal_feat = _f32(goal_feat); info_feat = _f32(info_feat)

    src = adj[0].astype(jnp.int32)
    dst = adj[1].astype(jnp.int32)
    order = jnp.argsort(src)
    src_s = src[order]
    dst_s = dst[order]
    counts = jnp.bincount(src, length=N)
    row_ptr = jnp.concatenate(
        [jnp.zeros((1,), jnp.int32), jnp.cumsum(counts).astype(jnp.int32)])
    srcs_pad = jnp.concatenate([src_s, jnp.full((EPAD,), N, jnp.int32)])
    dsts_pad = jnp.concatenate([dst_s, jnp.zeros((EPAD,), jnp.int32)])
    rp_pad = jnp.concatenate(
        [row_ptr, jnp.full((RPPAD - 1,), E, jnp.int32)])

    x = _enc_call(feat, goal_feat, info_feat,
                  enc_W1[:VIS], enc_W1[VIS:2 * VIS], enc_W1[2 * VIS:],
                  enc_b1.reshape(1, H),
                  enc_W2, enc_b2.reshape(1, H),
                  enc_W3, enc_b3.reshape(1, H))

    for Wg, ag in ((gat1_W, gat1_a), (gat2_W, gat2_a), (gat3_W, gat3_a),
                   (gat4_W, gat4_a), (gat5_W, gat5_a)):
        a2 = jnp.stack([ag[0, :H], ag[0, H:]], axis=1)
        a2p = jnp.pad(a2, ((0, 0), (0, 126)))
        h, av = _gat_pre_call(x, goal_feat, info_feat,
                              Wg[:H], Wg[H:H + VIS], Wg[H + VIS:], a2p)
        asrc = av[:, 0]
        adst = av[:, 1]
        x = _sc_gat_call(h, asrc, adst, srcs_pad, dsts_pad, rp_pad)

    w3p = jnp.pad(val_W3, ((0, 0), (0, 127)))
    b3p = jnp.pad(val_b3, (0, 127)).reshape(1, 128)
    out = _dec_call(x, goal_feat, info_feat,
                    val_W1[:VIS], val_W1[VIS:2 * VIS], val_W1[2 * VIS:],
                    val_b1.reshape(1, H),
                    val_W2, val_b2.reshape(1, H),
                    w3p, b3p)
    return out[:, :1]


# speedup vs baseline: 1.0387x; 1.0387x over previous
import functools

import jax
import jax.numpy as jnp
from jax import lax
from jax.experimental import pallas as pl
from jax.experimental.pallas import tpu as pltpu
from jax.experimental.pallas import tpu_sc as plsc

N = 10000
E = 160000
VIS = 512
H = 512
BN = 1000
GRID = N // BN

NC = 2
NS = 16
NW = NC * NS
TASK = 40
NTASK = N // TASK
GB = 96
EPAD = 256
RPPAD = 64


def _f32(x):
    return x.astype(jnp.float32)


def _enc_body(feat, goal, info, w1a, w1b, w1c, b1, w2, b2, w3, b3, out):
    x = jnp.dot(feat[...], w1a[...], preferred_element_type=jnp.float32)
    x += jnp.dot(goal[...], w1b[...], preferred_element_type=jnp.float32)
    x += jnp.dot(info[...], w1c[...], preferred_element_type=jnp.float32)
    x = jnp.maximum(x + b1[...], 0.0)
    x = jnp.maximum(jnp.dot(x, w2[...], preferred_element_type=jnp.float32) + b2[...], 0.0)
    out[...] = jnp.dot(x, w3[...], preferred_element_type=jnp.float32) + b3[...]


def _gat_pre_body(x, goal, info, wx, wg, wi, a2p, h_out, av_out):
    h = jnp.dot(x[...], wx[...], preferred_element_type=jnp.float32)
    h += jnp.dot(goal[...], wg[...], preferred_element_type=jnp.float32)
    h += jnp.dot(info[...], wi[...], preferred_element_type=jnp.float32)
    h_out[...] = h
    av_out[...] = jnp.dot(h, a2p[...], preferred_element_type=jnp.float32)


def _dec_body(x, goal, info, w1a, w1b, w1c, b1, w2, b2, w3p, b3p, out):
    y = jnp.dot(x[...], w1a[...], preferred_element_type=jnp.float32)
    y += jnp.dot(goal[...], w1b[...], preferred_element_type=jnp.float32)
    y += jnp.dot(info[...], w1c[...], preferred_element_type=jnp.float32)
    y = jnp.maximum(y + b1[...], 0.0)
    y = jnp.maximum(jnp.dot(y, w2[...], preferred_element_type=jnp.float32) + b2[...], 0.0)
    y = jnp.dot(y, w3p[...], preferred_element_type=jnp.float32) + b3p[...]
    out[...] = jax.nn.sigmoid(y)


def _row_spec(d):
    return pl.BlockSpec((BN, d), lambda i: (i, 0))


def _full_spec(shape):
    return pl.BlockSpec(shape, lambda i: tuple(0 for _ in shape))


def _enc_call(feat, goal, info, w1a, w1b, w1c, b1, w2, b2, w3, b3):
    return pl.pallas_call(
        _enc_body,
        grid=(GRID,),
        in_specs=[
            _row_spec(VIS), _row_spec(VIS), _row_spec(4),
            _full_spec((VIS, H)), _full_spec((VIS, H)), _full_spec((4, H)),
            _full_spec((1, H)),
            _full_spec((H, H)), _full_spec((1, H)),
            _full_spec((H, H)), _full_spec((1, H)),
        ],
        out_specs=_row_spec(H),
        out_shape=jax.ShapeDtypeStruct((N, H), jnp.float32),
    )(feat, goal, info, w1a, w1b, w1c, b1, w2, b2, w3, b3)


def _gat_pre_call(x, goal, info, wx, wg, wi, a2p):
    return pl.pallas_call(
        _gat_pre_body,
        grid=(GRID,),
        in_specs=[
            _row_spec(H), _row_spec(VIS), _row_spec(4),
            _full_spec((H, H)), _full_spec((VIS, H)), _full_spec((4, H)),
            _full_spec((H, 128)),
        ],
        out_specs=[_row_spec(H), _row_spec(128)],
        out_shape=[
            jax.ShapeDtypeStruct((N, H), jnp.float32),
            jax.ShapeDtypeStruct((N, 128), jnp.float32),
        ],
    )(x, goal, info, wx, wg, wi, a2p)


def _dec_call(x, goal, info, w1a, w1b, w1c, b1, w2, b2, w3p, b3p):
    return pl.pallas_call(
        _dec_body,
        grid=(GRID,),
        in_specs=[
            _row_spec(H), _row_spec(VIS), _row_spec(4),
            _full_spec((H, H)), _full_spec((VIS, H)), _full_spec((4, H)),
            _full_spec((1, H)),
            _full_spec((H, H)), _full_spec((1, H)),
            _full_spec((H, 128)), _full_spec((1, 128)),
        ],
        out_specs=_row_spec(128),
        out_shape=jax.ShapeDtypeStruct((N, 128), jnp.float32),
    )(x, goal, info, w1a, w1b, w1c, b1, w2, b2, w3p, b3p)


NACC = H // 16


def _sc_gat_body(h_hbm, asrc_hbm, adst_hbm, srcs_hbm, dsts_hbm, rp_hbm,
                 x_hbm,
                 rpv, evbuf, hp2,
                 sbuf0, cbuf0, dbuf0, asg0, adg0, rows0, semI0, semG0,
                 sbuf1, cbuf1, dbuf1, asg1, adg1, rows1, semI1, semG1):
    wid = lax.axis_index("s") * NC + lax.axis_index("c")
    slots = ((sbuf0, cbuf0, dbuf0, asg0, adg0, rows0, semI0, semG0),
             (sbuf1, cbuf1, dbuf1, asg1, adg1, rows1, semI1, semG1))

    zero16 = jnp.zeros((16,), jnp.float32)
    iota = lax.iota(jnp.int32, 16)
    lane0 = iota == 0
    zsel = jnp.zeros((16,), jnp.int32)

    def do_task(t):
        r0 = t * TASK

        def flush(cur, rs, acc):
            @pl.when(jnp.logical_and(cur >= r0, cur < r0 + TASK))
            def _():
                inv = 1.0 / jnp.full((16,), rs + 1e-16, jnp.float32)
                loc = cur - r0
                for c in range(NACC):
                    val = jnp.maximum(acc[c] * inv, 0.0)
                    hp2[pl.ds(loc * H + c * 16, 16)] = val

        def edge_step(rw, k, sk, ek, state):
            cur, rs, acc = state
            is_new = sk != cur

            @pl.when(is_new)
            def _():
                flush(cur, rs, acc)

            keepv = jnp.full((16,), jnp.where(is_new, 0.0, 1.0), jnp.float32)
            ekv = jnp.full((16,), ek, jnp.float32)
            nacc = tuple(acc[c] * keepv + ekv * rw[k, pl.ds(c * 16, 16)]
                         for c in range(NACC))
            nrs = jnp.where(is_new, ek, rs + ek)
            return sk, nrs, nacc

        pltpu.sync_copy(rp_hbm.at[pl.ds(r0, RPPAD)], rpv)

        def zrow(z, _):
            for c in range(NACC):
                hp2[pl.ds(z * H + c * 16, 16)] = zero16
            return 0
        lax.fori_loop(0, TASK, zrow, 0)

        p0 = rpv[pl.ds(0, 16)][0]
        p1 = rpv[pl.ds(TASK - 8, 16)][8]
        b0 = (p0 // 16) * 16
        nb = (p1 - b0 + GB - 1) // GB
        nb2 = ((nb + 1) // 2) * 2

        def issue_idx(g, s):
            sb, cb, db, _, _, _, sI, _ = slots[s]
            @pl.when(g < nb2)
            def _():
                pos = b0 + g * GB
                pltpu.async_copy(srcs_hbm.at[pl.ds(pos, GB)], sb, sI)
                pltpu.async_copy(dsts_hbm.at[pl.ds(pos, GB)], db, sI)

        def start_gather(g, s):
            sb, cb, db, ag, dg, rw, sI, sG = slots[s]
            @pl.when(g < nb2)
            def _():
                pltpu.make_async_copy(srcs_hbm.at[pl.ds(0, GB)], sb, sI).wait()
                pltpu.make_async_copy(dsts_hbm.at[pl.ds(0, GB)], db, sI).wait()
                for j in range(GB // 16):
                    cb[pl.ds(j * 16, 16)] = jnp.minimum(
                        sb[pl.ds(j * 16, 16)], N - 1)
                pltpu.async_copy(h_hbm.at[db], rw, sG)
                pltpu.async_copy(asrc_hbm.at[cb], ag, sG)
                pltpu.async_copy(adst_hbm.at[db], dg, sG)

        def process(s, state):
            sb, cb, db, ag, dg, rw, sI, sG = slots[s]
            pltpu.make_async_copy(h_hbm.at[db], rw, sG).wait()
            pltpu.make_async_copy(asrc_hbm.at[cb], ag, sG).wait()
            pltpu.make_async_copy(adst_hbm.at[db], dg, sG).wait()
            for j in range(GB // 16):
                az = ag[pl.ds(j * 16, 16)] + dg[pl.ds(j * 16, 16)]
                evbuf[pl.ds(j * 16, 16)] = jnp.exp(
                    jnp.minimum(-az, -0.2 * az))

            def sub(j, st):
                sv16 = sb[pl.ds(j * 16, 16)]
                ev16 = evbuf[pl.ds(j * 16, 16)]
                for k16 in range(16):
                    st = edge_step(rw, j * 16 + k16, sv16[k16], ev16[k16], st)
                return st
            return lax.fori_loop(0, GB // 16, sub, state)

        issue_idx(0, 0)
        issue_idx(1, 1)
        start_gather(0, 0)

        def pair(m, state):
            g = m * 2
            start_gather(g + 1, 1)
            state = process(0, state)
            issue_idx(g + 2, 0)
            start_gather(g + 2, 0)
            state = process(1, state)
            issue_idx(g + 3, 1)
            return state

        acc0 = tuple(zero16 for _ in range(NACC))
        state = lax.fori_loop(0, nb2 // 2, pair,
                              (jnp.int32(-1), jnp.float32(0.0), acc0))
        flush(state[0], state[1], state[2])

        pltpu.sync_copy(hp2, x_hbm.at[pl.ds(r0 * H, TASK * H)])

    nt_w = jnp.where(wid < (NTASK % NW), NTASK // NW + 1, NTASK // NW)

    def task_loop(i, _):
        do_task(wid + i * NW)
        return 0
    lax.fori_loop(0, nt_w, task_loop, 0)


def _sc_gat_call(h, asrc, adst, srcs_pad, dsts_pad, rp_pad):
    mesh = plsc.VectorSubcoreMesh(core_axis_name="c", subcore_axis_name="s")
    kfn = functools.partial(
        pl.kernel,
        out_type=jax.ShapeDtypeStruct((N * H,), jnp.float32),
        mesh=mesh,
        compiler_params=pltpu.CompilerParams(needs_layout_passes=False),
        scratch_types=[
            pltpu.VMEM((RPPAD,), jnp.int32),
            pltpu.VMEM((GB,), jnp.float32),
            pltpu.VMEM((TASK * H,), jnp.float32),
        ] + 2 * [
            pltpu.VMEM((GB,), jnp.int32),
            pltpu.VMEM((GB,), jnp.int32),
            pltpu.VMEM((GB,), jnp.int32),
            pltpu.VMEM((GB,), jnp.float32),
            pltpu.VMEM((GB,), jnp.float32),
            pltpu.VMEM((GB, H), jnp.float32),
            pltpu.SemaphoreType.DMA,
            pltpu.SemaphoreType.DMA,
        ],
    )(_sc_gat_body)
    return kfn(h, asrc, adst, srcs_pad, dsts_pad, rp_pad).reshape(N, H)


def kernel(feat, goal_feat, info_feat, adj,
           enc_W1, enc_b1, enc_W2, enc_b2, enc_W3, enc_b3,
           val_W1, val_b1, val_W2, val_b2, val_W3, val_b3,
           gat1_W, gat1_a, gat2_W, gat2_a, gat3_W, gat3_a,
           gat4_W, gat4_a, gat5_W, gat5_a):
    feat = _f32(feat); goal_feat = _f32(goal_feat); info_feat = _f32(info_feat)

    src = adj[0].astype(jnp.int32)
    dst = adj[1].astype(jnp.int32)
    keys = jnp.sort(src * 16384 + dst)
    src_s = keys // 16384
    dst_s = keys - src_s * 16384
    counts = jnp.bincount(src, length=N)
    row_ptr = jnp.concatenate(
        [jnp.zeros((1,), jnp.int32), jnp.cumsum(counts).astype(jnp.int32)])
    srcs_pad = jnp.concatenate([src_s, jnp.full((EPAD,), N, jnp.int32)])
    dsts_pad = jnp.concatenate([dst_s, jnp.zeros((EPAD,), jnp.int32)])
    rp_pad = jnp.concatenate(
        [row_ptr, jnp.full((RPPAD - 1,), E, jnp.int32)])

    x = _enc_call(feat, goal_feat, info_feat,
                  enc_W1[:VIS], enc_W1[VIS:2 * VIS], enc_W1[2 * VIS:],
                  enc_b1.reshape(1, H),
                  enc_W2, enc_b2.reshape(1, H),
                  enc_W3, enc_b3.reshape(1, H))

    for Wg, ag in ((gat1_W, gat1_a), (gat2_W, gat2_a), (gat3_W, gat3_a),
                   (gat4_W, gat4_a), (gat5_W, gat5_a)):
        a2 = jnp.stack([ag[0, :H], ag[0, H:]], axis=1)
        a2p = jnp.pad(a2, ((0, 0), (0, 126)))
        h, av = _gat_pre_call(x, goal_feat, info_feat,
                              Wg[:H], Wg[H:H + VIS], Wg[H + VIS:], a2p)
        asrc = av[:, 0]
        adst = av[:, 1]
        x = _sc_gat_call(h, asrc, adst, srcs_pad, dsts_pad, rp_pad)

    w3p = jnp.pad(val_W3, ((0, 0), (0, 127)))
    b3p = jnp.pad(val_b3, (0, 127)).reshape(1, 128)
    out = _dec_call(x, goal_feat, info_feat,
                    val_W1[:VIS], val_W1[VIS:2 * VIS], val_W1[2 * VIS:],
                    val_b1.reshape(1, H),
                    val_W2, val_b2.reshape(1, H),
                    w3p, b3p)
    return out[:, :1]
